# baseline (speedup 1.0000x reference)
import functools

import jax
import jax.numpy as jnp
from jax import lax
from jax.experimental import pallas as pl
from jax.experimental.pallas import tpu as pltpu
from jax.experimental.pallas import tpu_sc as plsc

L = 16
N = 1024
NV = N // L
CH = 3
ROW_W = N * CH
BITS = 6
BINS = 1 << BITS
PASSES = 6
CHUNK = N // L
PN = N + L
G = 4


def _sc_body(pts_hbm, out_hbm,
             bi00, bi01, bo00, bo01, bi10, bi11, bo10, bo11,
             ka0, va0, kb0, vb0, hist0, occ0,
             ka1, va1, kb1, vb1, hist1, occ1,
             si00, si01, so00, so01, si10, si11, so10, so11,
             *, nc, rows_per_w):
    wid = lax.axis_index("s") * nc + lax.axis_index("c")
    base = wid * rows_per_w
    lane = lax.broadcasted_iota(jnp.int32, (L,), 0)
    lane65 = lane * (CHUNK + 1)
    ones = jnp.ones((L,), jnp.int32)
    sign = jnp.full((L,), -(2 ** 31), jnp.int32)
    six = jnp.full((L,), 6, jnp.int32)
    ten = jnp.full((L,), 10, jnp.int32)

    def pad(pos):
        return pos + lax.shift_right_logical(pos, six)

    in_slots = ((bi00, bi01, si00, si01), (bi10, bi11, si10, si11))
    out_slots = ((bo00, bo01, so00, so01), (bo10, bo11, so10, so11))

    def start_in(slot, g):
        b0, b1, s0, s1 = in_slots[slot]
        pltpu.async_copy(pts_hbm.at[base + 2 * g], b0, s0)
        pltpu.async_copy(pts_hbm.at[base + 2 * g + 1], b1, s1)

    def wait_in(slot, g):
        b0, b1, s0, s1 = in_slots[slot]
        pltpu.make_async_copy(pts_hbm.at[base + 2 * g], b0, s0).wait()
        pltpu.make_async_copy(pts_hbm.at[base + 2 * g + 1], b1, s1).wait()

    def start_out(slot, g):
        b0, b1, s0, s1 = out_slots[slot]
        pltpu.async_copy(b0, out_hbm.at[base + 2 * g], s0)
        pltpu.async_copy(b1, out_hbm.at[base + 2 * g + 1], s1)

    def wait_out(slot, g):
        b0, b1, s0, s1 = out_slots[slot]
        pltpu.make_async_copy(b0, out_hbm.at[base + 2 * g], s0).wait()
        pltpu.make_async_copy(b1, out_hbm.at[base + 2 * g + 1], s1).wait()

    def compute(slot):
        bin0, bin1, _, _ = in_slots[slot]
        bout0, bout1, _, _ = out_slots[slot]
        slots = ((bin0, bout0, ka0, va0), (bin1, bout1, ka1, va1))

        @plsc.parallel_loop(0, NV, unroll=8)
        def _build(v):
            i = lane + v * L
            pa = pad(i)
            i3 = i * CH
            for (b_in, _, ka, va) in slots:
                x = plsc.load_gather(b_in, [i3])
                k = plsc.bitcast(x, jnp.int32)
                ks = jnp.where(k < 0, ~k, k ^ sign)
                plsc.store_scatter(ka, [pa], ks)
                plsc.store_scatter(va, [pa], i)

        for p in range(PASSES):
            if p % 2 == 0:
                srcs = [(ka0, va0, kb0, vb0, hist0, occ0),
                        (ka1, va1, kb1, vb1, hist1, occ1)]
            else:
                srcs = [(kb0, vb0, ka0, va0, hist0, occ0),
                        (kb1, vb1, ka1, va1, hist1, occ1)]
            shift = jnp.full((L,), p * BITS, jnp.int32)

            @plsc.parallel_loop(0, BINS, unroll=8)
            def _zero(v):
                z = jnp.zeros((L,), jnp.int32)
                hist0[pl.ds(v * L, L)] = z
                hist1[pl.ds(v * L, L)] = z

            def histo(g, c):
                j0 = g * G
                for (src_k, _, _, _, hi, ob) in srcs:
                    ds_ = []
                    addrs = []
                    for u in range(G):
                        k = plsc.load_gather(src_k, [lane65 + (j0 + u)])
                        d = lax.shift_right_logical(k, shift) & (BINS - 1)
                        ds_.append(d)
                        addrs.append(d * L + lane)
                    pre = [plsc.load_gather(hi, [a]) for a in addrs]
                    for u in range(G):
                        oc = pre[u]
                        for w in range(u):
                            oc = oc + jnp.where(ds_[u] == ds_[w], 1, 0)
                        pk = addrs[u] | lax.shift_left(oc, ten)
                        plsc.store_scatter(ob, [lane65 + (j0 + u)], pk)
                    for u in range(G):
                        plsc.addupdate_scatter(hi, [addrs[u]], ones)
                return c

            lax.fori_loop(0, NV // G, histo, 0, unroll=4)

            @plsc.parallel_loop(0, BINS, unroll=4,
                                carry=(jnp.int32(0), jnp.int32(0)))
            def _scan(v, carry):
                ca, cb = carry
                h0 = hist0[pl.ds(v * L, L)]
                h1 = hist1[pl.ds(v * L, L)]
                inc0 = plsc.cumsum(h0)
                inc1 = plsc.cumsum(h1)
                hist0[pl.ds(v * L, L)] = inc0 - h0 + ca
                hist1[pl.ds(v * L, L)] = inc1 - h1 + cb
                return (ca + inc0[L - 1], cb + inc1[L - 1])

            @plsc.parallel_loop(0, NV, unroll=4)
            def _permute(j):
                s = lane65 + j
                for (src_k, src_v, dst_k, dst_v, hi, ob) in srcs:
                    k = plsc.load_gather(src_k, [s])
                    v = plsc.load_gather(src_v, [s])
                    pk = plsc.load_gather(ob, [s])
                    addr = pk & (BINS * L - 1)
                    oc = lax.shift_right_logical(pk, ten)
                    b_ = plsc.load_gather(hi, [addr])
                    pa = pad(b_ + oc)
                    plsc.store_scatter(dst_k, [pa], k)
                    plsc.store_scatter(dst_v, [pa], v)

        @plsc.parallel_loop(0, NV, unroll=4)
        def _gather_out(j):
            rr = lane + j * L
            par = pad(rr)
            r3 = rr * CH
            for (b_in, b_out, _, va) in slots:
                v = plsc.load_gather(va, [par])
                v3 = v * CH
                for ch in range(CH):
                    x = plsc.load_gather(b_in, [v3 + ch])
                    plsc.store_scatter(b_out, [r3 + ch], x)

    npair = rows_per_w // 2
    start_in(0, 0)

    def step(g2, c):
        gA = 2 * g2
        gB = gA + 1
        start_in(1, gB)
        wait_in(0, gA)

        @pl.when(g2 > 0)
        def _():
            wait_out(0, gA - 2)

        compute(0)
        start_out(0, gA)

        @pl.when(g2 + 1 < npair // 2)
        def _():
            start_in(0, gA + 2)

        wait_in(1, gB)

        @pl.when(g2 > 0)
        def _():
            wait_out(1, gB - 2)

        compute(1)
        start_out(1, gB)
        return c

    lax.fori_loop(0, npair // 2, step, 0)
    wait_out(0, npair - 2)
    wait_out(1, npair - 1)


def kernel(point_set, field_dims=3):
    b = 1
    for s in point_set.shape[:-2]:
        b *= s
    pts = point_set.reshape(b, ROW_W)
    info = plsc.get_sparse_core_info()
    nc = info.num_cores
    nw = nc * info.num_subcores
    rows_per_w = b // nw
    mesh = plsc.VectorSubcoreMesh(core_axis_name="c", subcore_axis_name="s")
    body = functools.partial(_sc_body, nc=nc, rows_per_w=rows_per_w)
    fbuf = pltpu.VMEM((ROW_W,), jnp.float32)
    ibuf = pltpu.VMEM((PN,), jnp.int32)
    hbuf = pltpu.VMEM((BINS * L,), jnp.int32)
    dma = pltpu.SemaphoreType.DMA
    out = pl.kernel(
        body,
        out_type=jax.ShapeDtypeStruct((b, ROW_W), jnp.float32),
        mesh=mesh,
        compiler_params=pltpu.CompilerParams(needs_layout_passes=False),
        scratch_types=[fbuf] * 8
        + [ibuf, ibuf, ibuf, ibuf, hbuf, ibuf]
        + [ibuf, ibuf, ibuf, ibuf, hbuf, ibuf]
        + [dma] * 8,
    )(pts)
    return out.reshape(point_set.shape)
